# baseline (speedup 1.0000x reference)
import functools

import jax
import jax.numpy as jnp
from jax import lax
from jax.experimental import pallas as pl
from jax.experimental.pallas import tpu as pltpu
from jax.experimental.pallas import tpu_sc as plsc

NUM_EMBEDDINGS = 100001
EMBED_DIM = 32
BATCH = 16384

_CHUNK = 128


def _make_gather():
    info = plsc.get_sparse_core_info()
    nc, ns = info.num_cores, info.num_subcores
    nw = nc * ns
    b_per_w = BATCH // nw
    n_chunks = b_per_w // _CHUNK
    mesh = plsc.VectorSubcoreMesh(core_axis_name="c", subcore_axis_name="s")

    @functools.partial(
        pl.kernel,
        mesh=mesh,
        out_type=jax.ShapeDtypeStruct((BATCH, EMBED_DIM), jnp.float32),
        scratch_types=[
            pltpu.VMEM((n_chunks, _CHUNK), jnp.int32),
            pltpu.VMEM((b_per_w, EMBED_DIM), jnp.float32),
            pltpu.SemaphoreType.DMA,
        ],
        compiler_params=pltpu.CompilerParams(use_tc_tiling_on_sc=False),
    )
    def gather_kernel(table_hbm, idx_hbm, out_hbm, idx_v, rows_v, sem):
        wid = lax.axis_index("s") * nc + lax.axis_index("c")
        pltpu.sync_copy(idx_hbm.at[wid], idx_v)
        copies = [
            pltpu.async_copy(
                table_hbm.at[idx_v.at[j]],
                rows_v.at[pl.ds(j * _CHUNK, _CHUNK)],
                sem,
            )
            for j in range(n_chunks)
        ]
        for c in copies:
            c.wait()
        pltpu.sync_copy(rows_v, out_hbm.at[pl.ds(wid * b_per_w, b_per_w)])

    return gather_kernel, nw, n_chunks


def kernel(titles, embedding_table):
    gather_kernel, nw, n_chunks = _make_gather()
    idx = titles.reshape(nw, n_chunks, _CHUNK)
    return gather_kernel(embedding_table, idx)


# speedup vs baseline: 1.2224x; 1.2224x over previous
import functools

import jax
import jax.numpy as jnp
from jax import lax
from jax.experimental import pallas as pl
from jax.experimental.pallas import tpu as pltpu
from jax.experimental.pallas import tpu_sc as plsc

NUM_EMBEDDINGS = 100001
EMBED_DIM = 32
BATCH = 16384

_G = 16


def _make_gather():
    info = plsc.get_sparse_core_info()
    nc, ns = info.num_cores, info.num_subcores
    nw = nc * ns
    b_per_w = BATCH // nw
    n_groups = b_per_w // _G
    mesh = plsc.VectorSubcoreMesh(core_axis_name="c", subcore_axis_name="s")

    @functools.partial(
        pl.kernel,
        mesh=mesh,
        out_type=jax.ShapeDtypeStruct((BATCH, EMBED_DIM), jnp.float32),
        scratch_types=[
            pltpu.VMEM((b_per_w,), jnp.int32),
            pltpu.VMEM((b_per_w, EMBED_DIM), jnp.float32),
            pltpu.SemaphoreType.DMA,
        ],
    )
    def gather_kernel(table_hbm, idx_hbm, out_hbm, idx_v, rows_v, sem):
        wid = lax.axis_index("s") * nc + lax.axis_index("c")
        base = wid * b_per_w
        pltpu.sync_copy(idx_hbm.at[pl.ds(base, b_per_w)], idx_v)

        def enqueue(g):
            vec = idx_v[pl.ds(g * _G, _G)]
            for j in range(_G):
                pltpu.async_copy(
                    table_hbm.at[vec[j]], rows_v.at[g * _G + j], sem
                )

        def drain(g):
            pltpu.make_async_copy(
                table_hbm.at[pl.ds(0, _G)], rows_v.at[pl.ds(g * _G, _G)], sem
            ).wait()

        enqueue(0)

        def step(g, carry):
            enqueue(g)
            drain(g - 1)
            return carry

        lax.fori_loop(1, n_groups, step, 0)
        drain(n_groups - 1)
        pltpu.sync_copy(rows_v, out_hbm.at[pl.ds(base, b_per_w)])

    return gather_kernel


def kernel(titles, embedding_table):
    gather_kernel = _make_gather()
    return gather_kernel(embedding_table, titles)


# speedup vs baseline: 1.3465x; 1.1014x over previous
import functools

import jax
import jax.numpy as jnp
from jax import lax
from jax.experimental import pallas as pl
from jax.experimental.pallas import tpu as pltpu
from jax.experimental.pallas import tpu_sc as plsc

NUM_EMBEDDINGS = 100001
EMBED_DIM = 32
BATCH = 16384

_G = 16
_NBUF = 4


def _make_gather():
    info = plsc.get_sparse_core_info()
    nc, ns = info.num_cores, info.num_subcores
    nw = nc * ns
    b_per_w = BATCH // nw
    n_groups = b_per_w // _G
    mesh = plsc.VectorSubcoreMesh(core_axis_name="c", subcore_axis_name="s")

    @functools.partial(
        pl.kernel,
        mesh=mesh,
        out_type=jax.ShapeDtypeStruct((BATCH, EMBED_DIM), jnp.float32),
        scratch_types=[
            pltpu.VMEM((b_per_w,), jnp.int32),
            pltpu.VMEM((b_per_w, EMBED_DIM), jnp.float32),
            pltpu.SemaphoreType.DMA,
            pltpu.SemaphoreType.DMA,
        ],
    )
    def gather_kernel(table_hbm, idx_hbm, out_hbm, idx_v, rows_v, gsem, wsem):
        wid = lax.axis_index("s") * nc + lax.axis_index("c")
        base = wid * b_per_w
        pltpu.sync_copy(idx_hbm.at[pl.ds(base, b_per_w)], idx_v)

        def enqueue(g):
            vec = idx_v[pl.ds(g * _G, _G)]
            for j in range(_G):
                pltpu.async_copy(
                    table_hbm.at[vec[j]], rows_v.at[g * _G + j], gsem
                )

        def finish(g):
            pltpu.make_async_copy(
                table_hbm.at[pl.ds(0, _G)], rows_v.at[pl.ds(g * _G, _G)], gsem
            ).wait()
            pltpu.async_copy(
                rows_v.at[pl.ds(g * _G, _G)],
                out_hbm.at[pl.ds(base + g * _G, _G)],
                wsem,
            )

        for g in range(_NBUF):
            enqueue(g)

        def step(g, carry):
            enqueue(g)
            finish(g - _NBUF)
            return carry

        lax.fori_loop(_NBUF, n_groups, step, 0)
        for g in range(n_groups - _NBUF, n_groups):
            finish(g)
        pltpu.make_async_copy(
            rows_v, out_hbm.at[pl.ds(base, b_per_w)], wsem
        ).wait()

    return gather_kernel


def kernel(titles, embedding_table):
    gather_kernel = _make_gather()
    return gather_kernel(embedding_table, titles)
